# baseline (speedup 1.0000x reference)
import functools

import jax
import jax.numpy as jnp
from jax import lax
from jax.experimental import pallas as pl
from jax.experimental.pallas import tpu as pltpu
from jax.experimental.pallas import tpu_sc as plsc

B = 16384
D = 64
YD = 16
KG = 20
KT = 50
NC = 2
NS = 16
NW = NC * NS
R = B // NW
C = 32
CM = 512


def _sc_pool(genre_table, tag_table, g_flat, t_flat, dstg, dstt, zer):
    mesh = plsc.VectorSubcoreMesh(core_axis_name="c", subcore_axis_name="s")
    f32 = jnp.float32

    @functools.partial(
        pl.kernel,
        out_type=[
            jax.ShapeDtypeStruct((B, D), f32),
            jax.ShapeDtypeStruct((B, D), f32),
        ],
        mesh=mesh,
        scratch_types=[
            pltpu.VMEM((C * KT, D), f32),
            pltpu.VMEM_SHARED((NS, C, D), f32),
            pltpu.VMEM_SHARED((NS, C, D), f32),
            pltpu.VMEM_SHARED((1000, D), f32),
            pltpu.VMEM((C, D), f32),
            pltpu.VMEM((C * KG,), jnp.int32),
            pltpu.VMEM((C * KT,), jnp.int32),
            pltpu.VMEM((C * KG,), jnp.int32),
            pltpu.VMEM((C * KT,), jnp.int32),
        ],
        compiler_params=pltpu.CompilerParams(use_tc_tiling_on_sc=False),
    )
    def k(gt_hbm, tt_hbm, gid_hbm, tid_hbm, dstg_hbm, dstt_hbm, zer_hbm,
          g_out, t_out,
          buf, accg, acct, gt_spm, zeros_v, gidx, tidx, dstg_v, dstt_v):
        sid = lax.axis_index("s")
        wid = sid * NC + lax.axis_index("c")
        base0 = wid * R
        my_accg = accg.at[sid]
        my_acct = acct.at[sid]
        pltpu.sync_copy(dstg_hbm, dstg_v)
        pltpu.sync_copy(dstt_hbm, dstt_v)
        pltpu.sync_copy(zer_hbm, zeros_v)
        pltpu.sync_copy(gt_hbm, gt_spm)

        @pl.loop(0, R, step=C)
        def _(c0):
            base = base0 + c0
            pltpu.sync_copy(gid_hbm.at[pl.ds(base * KG, C * KG)], gidx)
            pltpu.sync_copy(tid_hbm.at[pl.ds(base * KT, C * KT)], tidx)
            pltpu.sync_copy(zeros_v, my_accg)
            pltpu.sync_copy(gt_spm.at[gidx], buf.at[pl.ds(0, C * KG)])
            pltpu.sync_copy(buf.at[pl.ds(0, C * KG)], my_accg.at[dstg_v], add=True)
            pltpu.sync_copy(my_accg, g_out.at[pl.ds(base, C)])
            pltpu.sync_copy(zeros_v, my_acct)
            pltpu.sync_copy(tt_hbm.at[tidx], buf)
            pltpu.sync_copy(buf, my_acct.at[dstt_v], add=True)
            pltpu.sync_copy(my_acct, t_out.at[pl.ds(base, C)])

    return k(genre_table, tag_table, g_flat, t_flat, dstg, dstt, zer)


def _sc_rows(movie_pad, year_pad, mid, yid):
    mesh = plsc.VectorSubcoreMesh(core_axis_name="c", subcore_axis_name="s")
    f32 = jnp.float32

    @functools.partial(
        pl.kernel,
        out_type=[
            jax.ShapeDtypeStruct((B, 128), f32),
            jax.ShapeDtypeStruct((B, 128), f32),
        ],
        mesh=mesh,
        scratch_types=[
            pltpu.VMEM((CM, 128), f32),
            pltpu.VMEM((CM,), jnp.int32),
            pltpu.VMEM((CM,), jnp.int32),
        ],
        compiler_params=pltpu.CompilerParams(use_tc_tiling_on_sc=True),
    )
    def k(mt_hbm, yt_hbm, mid_hbm, yid_hbm, m_out, y_out, buf, midx, yidx):
        wid = lax.axis_index("s") * NC + lax.axis_index("c")
        base0 = wid * R

        @pl.loop(0, R, step=CM)
        def _(c0):
            base = base0 + c0
            pltpu.sync_copy(mid_hbm.at[pl.ds(base, CM)], midx)
            pltpu.sync_copy(mt_hbm.at[midx], buf)
            pltpu.sync_copy(buf, m_out.at[pl.ds(base, CM)])
            pltpu.sync_copy(yid_hbm.at[pl.ds(base, CM)], yidx)
            pltpu.sync_copy(yt_hbm.at[yidx], buf)
            pltpu.sync_copy(buf, y_out.at[pl.ds(base, CM)])

    return k(movie_pad, year_pad, mid, yid)


def _mlp_body(m_ref, gs_ref, y_ref, ts_ref, gi_ref, ti_ref, par_ref,
              W1_ref, b1_ref, W2_ref, b2_ref, W3_ref, b3_ref, o_ref):
    f32 = jnp.float32
    gcnt = jnp.sum((gi_ref[...] != 0).astype(f32), axis=1, keepdims=True)
    tcnt = jnp.sum((ti_ref[...] != 0).astype(f32), axis=1, keepdims=True)
    g = gs_ref[...] / jnp.clip(gcnt, 1e-9, None)
    t = ts_ref[...] / jnp.clip(tcnt, 1e-9, None)
    W1 = W1_ref[...]
    m128 = m_ref[...]
    m = jnp.where(par_ref[...] > 0.5, m128[:, D:2 * D], m128[:, 0:D])
    x = (jnp.dot(m, W1[0:D], preferred_element_type=f32)
         + jnp.dot(g, W1[D:2 * D], preferred_element_type=f32)
         + jnp.dot(y_ref[:, :YD], W1[2 * D:2 * D + YD], preferred_element_type=f32)
         + jnp.dot(t, W1[2 * D + YD:], preferred_element_type=f32)
         + b1_ref[...])
    x = jnp.maximum(x, 0.0)
    h = jnp.maximum(jnp.dot(x, W2_ref[...], preferred_element_type=f32)
                    + b2_ref[...], 0.0)
    o_ref[...] = jnp.dot(h, W3_ref[...], preferred_element_type=f32) + b3_ref[...]


def _mlp(m_e, g_s, y_e, t_s, gidx, tidx, par, W1, b1, W2, b2, W3, b3):
    BT = 4096
    grid = (B // BT,)

    def rows(shape):
        return pl.BlockSpec((BT,) + shape[1:], lambda i: (i,) + (0,) * (len(shape) - 1))

    def whole(shape):
        return pl.BlockSpec(shape, lambda i: (0,) * len(shape))

    return pl.pallas_call(
        _mlp_body,
        grid=grid,
        in_specs=[
            rows((B, 128)), rows((B, D)), rows((B, 128)), rows((B, D)),
            rows((B, KG)), rows((B, KT)), rows((B, 1)),
            whole(W1.shape), whole(b1.shape), whole(W2.shape),
            whole(b2.shape), whole(W3.shape), whole(b3.shape),
        ],
        out_specs=rows((B, D)),
        out_shape=jax.ShapeDtypeStruct((B, D), jnp.float32),
    )(m_e, g_s, y_e, t_s, gidx, tidx, par, W1, b1, W2, b2, W3, b3)


def kernel(movie_id, padded_genre_indices, year_idx, padded_tag_indices,
           movie_table, genre_table, tag_table, year_table,
           W1, b1, W2, b2, W3, b3):
    mid = movie_id.astype(jnp.int32)
    yid = year_idx.astype(jnp.int32)
    gidx = padded_genre_indices.astype(jnp.int32)
    tidx = padded_tag_indices.astype(jnp.int32)
    g_flat = gidx.reshape(-1)
    t_flat = tidx.reshape(-1)
    dstg = jnp.arange(C * KG, dtype=jnp.int32) // KG
    dstt = jnp.arange(C * KT, dtype=jnp.int32) // KT
    zer = jnp.zeros((C, D), jnp.float32)
    movie2 = movie_table.reshape(movie_table.shape[0] // 2, 2 * D)
    year_pad = jnp.pad(year_table, ((0, 0), (0, 128 - YD)))
    par = (mid % 2).astype(jnp.float32).reshape(B, 1)
    g_s, t_s = _sc_pool(genre_table, tag_table, g_flat, t_flat, dstg, dstt, zer)
    m_e, y_e = _sc_rows(movie2, year_pad, mid // 2, yid)
    return _mlp(m_e, g_s, y_e, t_s, gidx, tidx, par,
                W1, b1.reshape(1, -1), W2, b2.reshape(1, -1),
                W3, b3.reshape(1, -1))


# speedup vs baseline: 1.0009x; 1.0009x over previous
import functools

import jax
import jax.numpy as jnp
from jax import lax
from jax.experimental import pallas as pl
from jax.experimental.pallas import tpu as pltpu
from jax.experimental.pallas import tpu_sc as plsc

B = 16384
D = 64
YD = 16
KG = 20
KT = 50
NC = 2
NS = 16
NW = NC * NS
R = B // NW
C = 32
CM = 512


def _sc_pool(genre_table, tag_table, g_flat, t_flat, dstg, dstt, zer):
    mesh = plsc.VectorSubcoreMesh(core_axis_name="c", subcore_axis_name="s")
    f32 = jnp.float32

    @functools.partial(
        pl.kernel,
        out_type=[
            jax.ShapeDtypeStruct((B, D), f32),
            jax.ShapeDtypeStruct((B, D), f32),
        ],
        mesh=mesh,
        scratch_types=[
            pltpu.VMEM((C * KT, D), f32),
            pltpu.VMEM_SHARED((NS, C, D), f32),
            pltpu.VMEM_SHARED((NS, C, D), f32),
            pltpu.VMEM_SHARED((1000, D), f32),
            pltpu.VMEM((C, D), f32),
            pltpu.VMEM((C * KG,), jnp.int32),
            pltpu.VMEM((C * KT,), jnp.int32),
            pltpu.VMEM((C * KG,), jnp.int32),
            pltpu.VMEM((C * KT,), jnp.int32),
        ],
        compiler_params=pltpu.CompilerParams(use_tc_tiling_on_sc=False),
    )
    def k(gt_hbm, tt_hbm, gid_hbm, tid_hbm, dstg_hbm, dstt_hbm, zer_hbm,
          g_out, t_out,
          buf, accg, acct, gt_spm, zeros_v, gidx, tidx, dstg_v, dstt_v):
        sid = lax.axis_index("s")
        wid = sid * NC + lax.axis_index("c")
        base0 = wid * R
        my_accg = accg.at[sid]
        my_acct = acct.at[sid]
        pltpu.sync_copy(dstg_hbm, dstg_v)
        pltpu.sync_copy(dstt_hbm, dstt_v)
        pltpu.sync_copy(zer_hbm, zeros_v)
        pltpu.sync_copy(gt_hbm, gt_spm)

        @pl.loop(0, R, step=C)
        def _(c0):
            base = base0 + c0
            pltpu.sync_copy(gid_hbm.at[pl.ds(base * KG, C * KG)], gidx)
            pltpu.sync_copy(tid_hbm.at[pl.ds(base * KT, C * KT)], tidx)
            pltpu.sync_copy(zeros_v, my_accg)
            pltpu.sync_copy(gt_spm.at[gidx], buf.at[pl.ds(0, C * KG)])
            pltpu.sync_copy(buf.at[pl.ds(0, C * KG)], my_accg.at[dstg_v], add=True)
            pltpu.sync_copy(my_accg, g_out.at[pl.ds(base, C)])
            pltpu.sync_copy(zeros_v, my_acct)
            pltpu.sync_copy(tt_hbm.at[tidx], buf)
            pltpu.sync_copy(buf, my_acct.at[dstt_v], add=True)
            pltpu.sync_copy(my_acct, t_out.at[pl.ds(base, C)])

    return k(genre_table, tag_table, g_flat, t_flat, dstg, dstt, zer)


def _sc_rows(movie_pad, year_pad, mid, yid):
    mesh = plsc.VectorSubcoreMesh(core_axis_name="c", subcore_axis_name="s")
    f32 = jnp.float32

    @functools.partial(
        pl.kernel,
        out_type=[
            jax.ShapeDtypeStruct((B, 128), f32),
            jax.ShapeDtypeStruct((B, 128), f32),
        ],
        mesh=mesh,
        scratch_types=[
            pltpu.VMEM((CM, 128), f32),
            pltpu.VMEM((CM,), jnp.int32),
            pltpu.VMEM((CM,), jnp.int32),
        ],
        compiler_params=pltpu.CompilerParams(use_tc_tiling_on_sc=True),
    )
    def k(mt_hbm, yt_hbm, mid_hbm, yid_hbm, m_out, y_out, buf, midx, yidx):
        wid = lax.axis_index("s") * NC + lax.axis_index("c")
        base0 = wid * R

        @pl.loop(0, R, step=CM)
        def _(c0):
            base = base0 + c0
            pltpu.sync_copy(mid_hbm.at[pl.ds(base, CM)], midx)
            pltpu.sync_copy(mt_hbm.at[midx], buf)
            pltpu.sync_copy(buf, m_out.at[pl.ds(base, CM)])
            pltpu.sync_copy(yid_hbm.at[pl.ds(base, CM)], yidx)
            pltpu.sync_copy(yt_hbm.at[yidx], buf)
            pltpu.sync_copy(buf, y_out.at[pl.ds(base, CM)])

    return k(movie_pad, year_pad, mid, yid)


def _mlp_body(m_ref, gs_ref, y_ref, ts_ref, gi_ref, ti_ref, par_ref,
              W1_ref, b1_ref, W2_ref, b2_ref, W3_ref, b3_ref, o_ref):
    f32 = jnp.float32
    gcnt = jnp.sum((gi_ref[...] != 0).astype(f32), axis=1, keepdims=True)
    tcnt = jnp.sum((ti_ref[...] != 0).astype(f32), axis=1, keepdims=True)
    g = gs_ref[...] / jnp.clip(gcnt, 1e-9, None)
    t = ts_ref[...] / jnp.clip(tcnt, 1e-9, None)
    W1 = W1_ref[...]
    m128 = m_ref[...]
    m = jnp.where(par_ref[...] > 0.5, m128[:, D:2 * D], m128[:, 0:D])
    x = (jnp.dot(m, W1[0:D], preferred_element_type=f32)
         + jnp.dot(g, W1[D:2 * D], preferred_element_type=f32)
         + jnp.dot(y_ref[:, :YD], W1[2 * D:2 * D + YD],
                   preferred_element_type=f32)
         + jnp.dot(t, W1[2 * D + YD:], preferred_element_type=f32)
         + b1_ref[...])
    x = jnp.maximum(x, 0.0)
    h = jnp.maximum(jnp.dot(x, W2_ref[...], preferred_element_type=f32)
                    + b2_ref[...], 0.0)
    o_ref[...] = jnp.dot(h, W3_ref[...], preferred_element_type=f32) + b3_ref[...]


def _mlp(m_e, g_s, y_e, t_s, gidx, tidx, par, W1, b1, W2, b2, W3, b3):
    BT = 2048
    grid = (B // BT,)

    def rows(shape):
        return pl.BlockSpec((BT,) + shape[1:], lambda i: (i,) + (0,) * (len(shape) - 1))

    def whole(shape):
        return pl.BlockSpec(shape, lambda i: (0,) * len(shape))

    return pl.pallas_call(
        _mlp_body,
        grid=grid,
        in_specs=[
            rows((B, 128)), rows((B, D)), rows((B, 128)), rows((B, D)),
            rows((B, KG)), rows((B, KT)), rows((B, 1)),
            whole(W1.shape), whole(b1.shape), whole(W2.shape),
            whole(b2.shape), whole(W3.shape), whole(b3.shape),
        ],
        out_specs=rows((B, D)),
        out_shape=jax.ShapeDtypeStruct((B, D), jnp.float32),
    )(m_e, g_s, y_e, t_s, gidx, tidx, par, W1, b1, W2, b2, W3, b3)


def kernel(movie_id, padded_genre_indices, year_idx, padded_tag_indices,
           movie_table, genre_table, tag_table, year_table,
           W1, b1, W2, b2, W3, b3):
    mid = movie_id.astype(jnp.int32)
    yid = year_idx.astype(jnp.int32)
    gidx = padded_genre_indices.astype(jnp.int32)
    tidx = padded_tag_indices.astype(jnp.int32)
    g_flat = gidx.reshape(-1)
    t_flat = tidx.reshape(-1)
    dstg = jnp.arange(C * KG, dtype=jnp.int32) // KG
    dstt = jnp.arange(C * KT, dtype=jnp.int32) // KT
    zer = jnp.zeros((C, D), jnp.float32)
    movie2 = movie_table.reshape(movie_table.shape[0] // 2, 2 * D)
    year_pad = jnp.pad(year_table, ((0, 0), (0, 128 - YD)))
    par = (mid % 2).astype(jnp.float32).reshape(B, 1)
    g_s, t_s = _sc_pool(genre_table, tag_table, g_flat, t_flat, dstg, dstt, zer)
    m_e, y_e = _sc_rows(movie2, year_pad, mid // 2, yid)
    return _mlp(m_e, g_s, y_e, t_s, gidx, tidx, par,
                W1, b1.reshape(1, -1), W2, b2.reshape(1, -1),
                W3, b3.reshape(1, -1))
